# baseline (speedup 1.0000x reference)
import functools

import jax
import jax.numpy as jnp
from jax import lax
from jax.experimental import pallas as pl
from jax.experimental.pallas import tpu as pltpu
from jax.experimental.pallas import tpu_sc as plsc

_K = 65536
_DIM = 128
_B = 16384
_PTR = 60000
_TAIL = _K - _PTR
_HEAD = _B - _TAIL
_MID = _PTR - _HEAD

_SC_INFO = plsc.get_sparse_core_info()
_NC = _SC_INFO.num_cores


_CH = 8192
_DEPTH = 6

_Z_CHUNKS = []
for _off in range(0, _TAIL, _CH):
    _Z_CHUNKS.append((0, _off, _PTR + _off, min(_CH, _TAIL - _off)))
for _off in range(0, _HEAD, _CH):
    _Z_CHUNKS.append((0, _TAIL + _off, _off, min(_CH, _HEAD - _off)))
for _off in range(0, _MID, _CH):
    _Z_CHUNKS.append((1, _HEAD + _off, _HEAD + _off, min(_CH, _MID - _off)))


def _z_body(z_new, z_buf, z_out, scratch, in_sem, out_sem):
    srcs = (z_new, z_buf)
    n_chunks = len(_Z_CHUNKS)

    def in_copy(k):
        s, so, _, n = _Z_CHUNKS[k]
        return pltpu.make_async_copy(
            srcs[s].at[pl.ds(so, n)],
            scratch.at[k % _DEPTH, pl.ds(0, n)],
            in_sem.at[k % _DEPTH])

    def out_copy(k):
        _, _, do, n = _Z_CHUNKS[k]
        return pltpu.make_async_copy(
            scratch.at[k % _DEPTH, pl.ds(0, n)],
            z_out.at[pl.ds(do, n)],
            out_sem.at[k % _DEPTH])

    for k in range(min(_DEPTH, n_chunks)):
        in_copy(k).start()
    for k in range(n_chunks):
        in_copy(k).wait()
        out_copy(k).start()
        if k + _DEPTH < n_chunks:
            out_copy(k).wait()
            in_copy(k + _DEPTH).start()
    for k in range(max(0, n_chunks - _DEPTH), n_chunks):
        out_copy(k).wait()


_SC_PIECE = 6144


def _split_task(so, do, n):
    pieces = -(-n // _SC_PIECE)
    step = -(-n // pieces // 8) * 8
    out = []
    off = 0
    while off < n:
        out.append((so + off, do + off, min(step, n - off)))
        off += step
    return out


def _sc_body(t_new, e_new, b_new, t_buf, e_buf, b_buf,
             t_out, e_out, b_out, fscr, iscr):
    wid = lax.axis_index("s") * _NC + lax.axis_index("c")
    pieces = []
    for new, buf, out, scr in ((t_new, t_buf, t_out, fscr),
                               (e_new, e_buf, e_out, fscr),
                               (b_new, b_buf, b_out, iscr)):
        for src, so, do, n in ((new, 0, _PTR, _TAIL),
                               (new, _TAIL, 0, _HEAD),
                               (buf, _HEAD, _HEAD, _MID)):
            for po, pd, pn in _split_task(so, do, n):
                pieces.append((src, po, out, pd, pn, scr))
    n_workers = _SC_INFO.num_cores * _SC_INFO.num_subcores
    for k, (src, so, dst, do, n, scr) in enumerate(pieces):
        @pl.when(wid == k % n_workers)
        def _():
            pltpu.sync_copy(src.at[pl.ds(so, n)], scr.at[pl.ds(0, n)])
            pltpu.sync_copy(scr.at[pl.ds(0, n)], dst.at[pl.ds(do, n)])


_sc_enqueue = functools.partial(
    pl.kernel,
    out_type=(
        jax.ShapeDtypeStruct((_K,), jnp.float32),
        jax.ShapeDtypeStruct((_K,), jnp.float32),
        jax.ShapeDtypeStruct((_K,), jnp.int32),
    ),
    mesh=plsc.VectorSubcoreMesh(core_axis_name="c", subcore_axis_name="s"),
    scratch_types=[
        pltpu.VMEM((_SC_PIECE,), jnp.float32),
        pltpu.VMEM((_SC_PIECE,), jnp.int32),
    ],
)(_sc_body)


def kernel(z_new, t_new, e_new, b_new, z_buf, t_buf, e_buf, b_buf):
    z = pl.pallas_call(
        _z_body,
        out_shape=jax.ShapeDtypeStruct((_K, _DIM), jnp.float32),
        in_specs=[pl.BlockSpec(memory_space=pltpu.HBM)] * 2,
        out_specs=pl.BlockSpec(memory_space=pltpu.HBM),
        scratch_shapes=[
            pltpu.VMEM((_DEPTH, _CH, _DIM), jnp.float32),
            pltpu.SemaphoreType.DMA((_DEPTH,)),
            pltpu.SemaphoreType.DMA((_DEPTH,)),
        ],
    )(z_new, z_buf)
    t, e, b = _sc_enqueue(t_new, e_new, b_new, t_buf, e_buf, b_buf)
    new_ptr = jnp.asarray((_PTR + _B) % _K, dtype=jnp.int32)
    new_size = jnp.asarray(min(_B, _K), dtype=jnp.int32)
    return (z, t, e, b, new_ptr, new_size)


# speedup vs baseline: 1.0006x; 1.0006x over previous
import functools

import jax
import jax.numpy as jnp
from jax import lax
from jax.experimental import pallas as pl
from jax.experimental.pallas import tpu as pltpu
from jax.experimental.pallas import tpu_sc as plsc

_K = 65536
_DIM = 128
_B = 16384
_PTR = 60000
_TAIL = _K - _PTR
_HEAD = _B - _TAIL
_MID = _PTR - _HEAD

_SC_INFO = plsc.get_sparse_core_info()
_NC = _SC_INFO.num_cores


_CH = 8192
_DEPTH = 6

_Z_CHUNKS = []
for _off in range(0, _TAIL, _CH):
    _Z_CHUNKS.append((0, _off, _PTR + _off, min(_CH, _TAIL - _off)))
for _off in range(0, _HEAD, _CH):
    _Z_CHUNKS.append((0, _TAIL + _off, _off, min(_CH, _HEAD - _off)))
for _off in range(0, _MID, _CH):
    _Z_CHUNKS.append((1, _HEAD + _off, _HEAD + _off, min(_CH, _MID - _off)))


def _z_body(z_new, z_buf, z_out, scratch, in_sem, out_sem):
    srcs = (z_new, z_buf)
    n_chunks = len(_Z_CHUNKS)

    def in_copy(k):
        s, so, _, n = _Z_CHUNKS[k]
        return pltpu.make_async_copy(
            srcs[s].at[pl.ds(so, n)],
            scratch.at[k % _DEPTH, pl.ds(0, n)],
            in_sem.at[k % _DEPTH])

    def out_copy(k):
        _, _, do, n = _Z_CHUNKS[k]
        return pltpu.make_async_copy(
            scratch.at[k % _DEPTH, pl.ds(0, n)],
            z_out.at[pl.ds(do, n)],
            out_sem.at[k % _DEPTH])

    for k in range(min(_DEPTH, n_chunks)):
        in_copy(k).start()
    for k in range(n_chunks):
        in_copy(k).wait()
        out_copy(k).start()
        if k + _DEPTH < n_chunks:
            out_copy(k).wait()
            in_copy(k + _DEPTH).start()
    for k in range(max(0, n_chunks - _DEPTH), n_chunks):
        out_copy(k).wait()


_SC_PIECE = 6144


def _split_task(so, do, n):
    pieces = -(-n // _SC_PIECE)
    step = -(-n // pieces // 8) * 8
    out = []
    off = 0
    while off < n:
        out.append((so + off, do + off, min(step, n - off)))
        off += step
    return out


def _sc_body(t_new, e_new, b_new, t_buf, e_buf, b_buf,
             t_out, e_out, b_out, fscr, iscr):
    wid = lax.axis_index("s") * _NC + lax.axis_index("c")
    pieces = []
    for new, buf, out, scr in ((t_new, t_buf, t_out, fscr),
                               (e_new, e_buf, e_out, fscr),
                               (b_new, b_buf, b_out, iscr)):
        for src, so, do, n in ((new, 0, _PTR, _TAIL),
                               (new, _TAIL, 0, _HEAD),
                               (buf, _HEAD, _HEAD, _MID)):
            for po, pd, pn in _split_task(so, do, n):
                pieces.append((src, po, out, pd, pn, scr))
    n_workers = _SC_INFO.num_cores * _SC_INFO.num_subcores
    for k, (src, so, dst, do, n, scr) in enumerate(pieces):
        @pl.when(wid == k % n_workers)
        def _():
            pltpu.sync_copy(src.at[pl.ds(so, n)], scr.at[pl.ds(0, n)])
            pltpu.sync_copy(scr.at[pl.ds(0, n)], dst.at[pl.ds(do, n)])


_sc_enqueue = functools.partial(
    pl.kernel,
    out_type=(
        jax.ShapeDtypeStruct((_K,), jnp.float32),
        jax.ShapeDtypeStruct((_K,), jnp.float32),
        jax.ShapeDtypeStruct((_K,), jnp.int32),
    ),
    mesh=plsc.VectorSubcoreMesh(core_axis_name="c", subcore_axis_name="s"),
    scratch_types=[
        pltpu.VMEM((_SC_PIECE,), jnp.float32),
        pltpu.VMEM((_SC_PIECE,), jnp.int32),
    ],
)(_sc_body)


def kernel(z_new, t_new, e_new, b_new, z_buf, t_buf, e_buf, b_buf):
    t, e, b = _sc_enqueue(t_new, e_new, b_new, t_buf, e_buf, b_buf)
    z = pl.pallas_call(
        _z_body,
        out_shape=jax.ShapeDtypeStruct((_K, _DIM), jnp.float32),
        in_specs=[pl.BlockSpec(memory_space=pltpu.HBM)] * 2,
        out_specs=pl.BlockSpec(memory_space=pltpu.HBM),
        scratch_shapes=[
            pltpu.VMEM((_DEPTH, _CH, _DIM), jnp.float32),
            pltpu.SemaphoreType.DMA((_DEPTH,)),
            pltpu.SemaphoreType.DMA((_DEPTH,)),
        ],
    )(z_new, z_buf)
    new_ptr = jnp.asarray((_PTR + _B) % _K, dtype=jnp.int32)
    new_size = jnp.asarray(min(_B, _K), dtype=jnp.int32)
    return (z, t, e, b, new_ptr, new_size)


# speedup vs baseline: 1.0405x; 1.0399x over previous
import functools

import jax
import jax.numpy as jnp
from jax import lax
from jax.experimental import pallas as pl
from jax.experimental.pallas import tpu as pltpu
from jax.experimental.pallas import tpu_sc as plsc

_K = 65536
_DIM = 128
_B = 16384
_PTR = 60000
_TAIL = _K - _PTR
_HEAD = _B - _TAIL
_MID = _PTR - _HEAD

_SC_INFO = plsc.get_sparse_core_info()
_SC_CORES = 1
_SC_WORKERS = _SC_CORES * _SC_INFO.num_subcores


_CH = 8192
_DEPTH = 6

_Z_CHUNKS = []
for _off in range(0, _TAIL, _CH):
    _Z_CHUNKS.append((0, _off, _PTR + _off, min(_CH, _TAIL - _off)))
for _off in range(0, _HEAD, _CH):
    _Z_CHUNKS.append((0, _TAIL + _off, _off, min(_CH, _HEAD - _off)))
for _off in range(0, _MID, _CH):
    _Z_CHUNKS.append((1, _HEAD + _off, _HEAD + _off, min(_CH, _MID - _off)))


def _z_body(z_new, z_buf, z_out, scratch, in_sem, out_sem):
    srcs = (z_new, z_buf)
    n_chunks = len(_Z_CHUNKS)

    def in_copy(k):
        s, so, _, n = _Z_CHUNKS[k]
        return pltpu.make_async_copy(
            srcs[s].at[pl.ds(so, n)],
            scratch.at[k % _DEPTH, pl.ds(0, n)],
            in_sem.at[k % _DEPTH])

    def out_copy(k):
        _, _, do, n = _Z_CHUNKS[k]
        return pltpu.make_async_copy(
            scratch.at[k % _DEPTH, pl.ds(0, n)],
            z_out.at[pl.ds(do, n)],
            out_sem.at[k % _DEPTH])

    for k in range(min(_DEPTH, n_chunks)):
        in_copy(k).start()
    for k in range(n_chunks):
        in_copy(k).wait()
        out_copy(k).start()
        if k + _DEPTH < n_chunks:
            out_copy(k).wait()
            in_copy(k + _DEPTH).start()
    for k in range(max(0, n_chunks - _DEPTH), n_chunks):
        out_copy(k).wait()


_SC_PIECE = 24576


def _split_task(so, do, n):
    pieces = -(-n // _SC_PIECE)
    step = -(-n // pieces // 8) * 8
    out = []
    off = 0
    while off < n:
        out.append((so + off, do + off, min(step, n - off)))
        off += step
    return out


def _sc_body(t_new, e_new, b_new, t_buf, e_buf, b_buf,
             t_out, e_out, b_out, fscr, iscr):
    wid = lax.axis_index("s") * _SC_CORES + lax.axis_index("c")
    pieces = []
    for new, buf, out, scr in ((t_new, t_buf, t_out, fscr),
                               (e_new, e_buf, e_out, fscr),
                               (b_new, b_buf, b_out, iscr)):
        for src, so, do, n in ((new, 0, _PTR, _TAIL),
                               (new, _TAIL, 0, _HEAD),
                               (buf, _HEAD, _HEAD, _MID)):
            for po, pd, pn in _split_task(so, do, n):
                pieces.append((src, po, out, pd, pn, scr))
    for k, (src, so, dst, do, n, scr) in enumerate(pieces):
        @pl.when(wid == k % _SC_WORKERS)
        def _():
            pltpu.sync_copy(src.at[pl.ds(so, n)], scr.at[pl.ds(0, n)])
            pltpu.sync_copy(scr.at[pl.ds(0, n)], dst.at[pl.ds(do, n)])


_sc_enqueue = functools.partial(
    pl.kernel,
    out_type=(
        jax.ShapeDtypeStruct((_K,), jnp.float32),
        jax.ShapeDtypeStruct((_K,), jnp.float32),
        jax.ShapeDtypeStruct((_K,), jnp.int32),
    ),
    mesh=plsc.VectorSubcoreMesh(core_axis_name="c", subcore_axis_name="s",
                                num_cores=_SC_CORES),
    scratch_types=[
        pltpu.VMEM((_SC_PIECE,), jnp.float32),
        pltpu.VMEM((_SC_PIECE,), jnp.int32),
    ],
)(_sc_body)


def kernel(z_new, t_new, e_new, b_new, z_buf, t_buf, e_buf, b_buf):
    t, e, b = _sc_enqueue(t_new, e_new, b_new, t_buf, e_buf, b_buf)
    z = pl.pallas_call(
        _z_body,
        out_shape=jax.ShapeDtypeStruct((_K, _DIM), jnp.float32),
        in_specs=[pl.BlockSpec(memory_space=pltpu.HBM)] * 2,
        out_specs=pl.BlockSpec(memory_space=pltpu.HBM),
        scratch_shapes=[
            pltpu.VMEM((_DEPTH, _CH, _DIM), jnp.float32),
            pltpu.SemaphoreType.DMA((_DEPTH,)),
            pltpu.SemaphoreType.DMA((_DEPTH,)),
        ],
    )(z_new, z_buf)
    new_ptr = jnp.asarray((_PTR + _B) % _K, dtype=jnp.int32)
    new_size = jnp.asarray(min(_B, _K), dtype=jnp.int32)
    return (z, t, e, b, new_ptr, new_size)


# speedup vs baseline: 1.6207x; 1.5575x over previous
import jax
import jax.numpy as jnp
from jax import lax
from jax.experimental import pallas as pl
from jax.experimental.pallas import tpu as pltpu

_K = 65536
_DIM = 128
_B = 16384
_PTR = 60000
_TAIL = _K - _PTR
_HEAD = _B - _TAIL
_MID = _PTR - _HEAD

_CH = 8192
_DEPTH = 6

_Z_CHUNKS = []
for _off in range(0, _TAIL, _CH):
    _Z_CHUNKS.append((0, _off, _PTR + _off, min(_CH, _TAIL - _off)))
for _off in range(0, _HEAD, _CH):
    _Z_CHUNKS.append((0, _TAIL + _off, _off, min(_CH, _HEAD - _off)))
for _off in range(0, _MID, _CH):
    _Z_CHUNKS.append((1, _HEAD + _off, _HEAD + _off, min(_CH, _MID - _off)))

_TB_R = _K // _DIM
_TBN_R = _B // _DIM
_ROW_SH = _TAIL // _DIM
_LANE_SH = _TAIL % _DIM


def _tb_scatter(pad, buf):
    a2 = pltpu.roll(pltpu.roll(pad, _TB_R - _ROW_SH, 0), _DIM - _LANE_SH, 1)
    b2 = pltpu.roll(a2, _TB_R - 1, 0)
    row = lax.broadcasted_iota(jnp.int32, (_TB_R, _DIM), 0)
    col = lax.broadcasted_iota(jnp.int32, (_TB_R, _DIM), 1)
    p = row * _DIM + col
    written = (p < _HEAD) | (p >= _PTR)
    return jnp.where(written, jnp.where(col < _DIM - _LANE_SH, a2, b2), buf)


def _body(z_new, z_buf, tn, en, bn, tb, eb, bb,
          z_out, t_out, e_out, b_out,
          zscr, z_in_sem, z_out_sem,
          tpad, epad, bpad, tbuf, ebuf, bbuf, tb_in_sem, tb_out_sem):
    srcs = (z_new, z_buf)
    n_chunks = len(_Z_CHUNKS)

    def in_copy(k):
        s, so, _, n = _Z_CHUNKS[k]
        return pltpu.make_async_copy(
            srcs[s].at[pl.ds(so, n)],
            zscr.at[k % _DEPTH, pl.ds(0, n)],
            z_in_sem.at[k % _DEPTH])

    def out_copy(k):
        _, _, do, n = _Z_CHUNKS[k]
        return pltpu.make_async_copy(
            zscr.at[k % _DEPTH, pl.ds(0, n)],
            z_out.at[pl.ds(do, n)],
            z_out_sem.at[k % _DEPTH])

    tb_ins = [
        pltpu.make_async_copy(tn, tpad.at[pl.ds(0, _TBN_R)], tb_in_sem.at[0]),
        pltpu.make_async_copy(en, epad.at[pl.ds(0, _TBN_R)], tb_in_sem.at[1]),
        pltpu.make_async_copy(bn, bpad.at[pl.ds(0, _TBN_R)], tb_in_sem.at[2]),
        pltpu.make_async_copy(tb, tbuf, tb_in_sem.at[3]),
        pltpu.make_async_copy(eb, ebuf, tb_in_sem.at[4]),
        pltpu.make_async_copy(bb, bbuf, tb_in_sem.at[5]),
    ]
    for c in tb_ins:
        c.start()
    for k in range(min(_DEPTH, n_chunks)):
        in_copy(k).start()

    for c in tb_ins:
        c.wait()
    tbuf[...] = _tb_scatter(tpad[...], tbuf[...])
    ebuf[...] = _tb_scatter(epad[...], ebuf[...])
    bbuf[...] = _tb_scatter(bpad[...], bbuf[...])
    tb_outs = [
        pltpu.make_async_copy(tbuf, t_out, tb_out_sem.at[0]),
        pltpu.make_async_copy(ebuf, e_out, tb_out_sem.at[1]),
        pltpu.make_async_copy(bbuf, b_out, tb_out_sem.at[2]),
    ]
    for c in tb_outs:
        c.start()

    for k in range(n_chunks):
        in_copy(k).wait()
        out_copy(k).start()
        if k + _DEPTH < n_chunks:
            out_copy(k).wait()
            in_copy(k + _DEPTH).start()
    for k in range(max(0, n_chunks - _DEPTH), n_chunks):
        out_copy(k).wait()
    for c in tb_outs:
        c.wait()


def kernel(z_new, t_new, e_new, b_new, z_buf, t_buf, e_buf, b_buf):
    z, t2, e2, b2 = pl.pallas_call(
        _body,
        out_shape=(
            jax.ShapeDtypeStruct((_K, _DIM), jnp.float32),
            jax.ShapeDtypeStruct((_TB_R, _DIM), jnp.float32),
            jax.ShapeDtypeStruct((_TB_R, _DIM), jnp.float32),
            jax.ShapeDtypeStruct((_TB_R, _DIM), jnp.int32),
        ),
        in_specs=[pl.BlockSpec(memory_space=pltpu.HBM)] * 8,
        out_specs=[pl.BlockSpec(memory_space=pltpu.HBM)] * 4,
        scratch_shapes=[
            pltpu.VMEM((_DEPTH, _CH, _DIM), jnp.float32),
            pltpu.SemaphoreType.DMA((_DEPTH,)),
            pltpu.SemaphoreType.DMA((_DEPTH,)),
            pltpu.VMEM((_TB_R, _DIM), jnp.float32),
            pltpu.VMEM((_TB_R, _DIM), jnp.float32),
            pltpu.VMEM((_TB_R, _DIM), jnp.int32),
            pltpu.VMEM((_TB_R, _DIM), jnp.float32),
            pltpu.VMEM((_TB_R, _DIM), jnp.float32),
            pltpu.VMEM((_TB_R, _DIM), jnp.int32),
            pltpu.SemaphoreType.DMA((6,)),
            pltpu.SemaphoreType.DMA((3,)),
        ],
    )(z_new, z_buf,
      t_new.reshape(_TBN_R, _DIM), e_new.reshape(_TBN_R, _DIM),
      b_new.reshape(_TBN_R, _DIM),
      t_buf.reshape(_TB_R, _DIM), e_buf.reshape(_TB_R, _DIM),
      b_buf.reshape(_TB_R, _DIM))
    new_ptr = jnp.asarray((_PTR + _B) % _K, dtype=jnp.int32)
    new_size = jnp.asarray(min(_B, _K), dtype=jnp.int32)
    return (z, t2.reshape(_K), e2.reshape(_K), b2.reshape(_K),
            new_ptr, new_size)
